# Initial kernel scaffold; baseline (speedup 1.0000x reference)
import jax
import jax.numpy as jnp
from jax.experimental import pallas as pl


def kernel(feat, index, w, b):
    raise NotImplementedError("write your pallas kernel here")


# speedup vs baseline: 94.3148x; 94.3148x over previous
import jax
import jax.numpy as jnp
from jax.experimental import pallas as pl

_FILTERS = 64
_C = 64
_D = _H = _W = 32
_PZ, _PY, _PX = 36, 34, 34
_PLANE = _PY * _PX
_NPAD = _PZ * _PLANE
_R0 = 2048
_TB = 2048
_G = 19
_L = _TB * _G
_FEXT = 42240

_OFFS = tuple((kk // 9 - 1) * _PLANE + ((kk // 3) % 3 - 1) * _PY + (kk % 3 - 1)
              for kk in range(27))


def _conv_body(fext_ref, w_ref, b_ref, mask_ref, out_ref):
    g = pl.program_id(0)
    base = _R0 + g * _TB
    acc = jnp.dot(fext_ref[pl.ds(base + _OFFS[0], _TB), :], w_ref[0],
                  preferred_element_type=jnp.float32)
    for kk in range(1, 27):
        acc += jnp.dot(fext_ref[pl.ds(base + _OFFS[kk], _TB), :], w_ref[kk],
                       preferred_element_type=jnp.float32)
    out_ref[...] = (acc + b_ref[...]) * mask_ref[...]


def kernel(feat, index, w, b):
    f = feat.reshape(_D, _H, _W, _C)
    fp = jnp.pad(f, ((2, 2), (1, 1), (1, 1), (0, 0))).reshape(_NPAD, _C)
    fext = jnp.pad(fp, ((0, _FEXT - _NPAD), (0, 0)))
    m = (index.reshape(_D, _H, _W) != 0).astype(feat.dtype)
    mp = jnp.pad(m, ((2, 2), (1, 1), (1, 1))).reshape(_NPAD)
    maskb = jnp.broadcast_to(mp[_R0:_R0 + _L, None], (_L, _FILTERS))
    w27 = w.reshape(27, _C, _FILTERS)

    out = pl.pallas_call(
        _conv_body,
        grid=(_G,),
        in_specs=[
            pl.BlockSpec((_FEXT, _C), lambda g: (0, 0)),
            pl.BlockSpec((27, _C, _FILTERS), lambda g: (0, 0, 0)),
            pl.BlockSpec((1, _FILTERS), lambda g: (0, 0)),
            pl.BlockSpec((_TB, _FILTERS), lambda g: (g, 0)),
        ],
        out_specs=pl.BlockSpec((_TB, _FILTERS), lambda g: (g, 0)),
        out_shape=jax.ShapeDtypeStruct((_L, _FILTERS), feat.dtype),
    )(fext, w27, b.reshape(1, _FILTERS), maskb)

    full = jnp.pad(out, ((_R0, _NPAD - _R0 - _L), (0, 0)))
    full = full.reshape(_PZ, _PY, _PX, _FILTERS)[2:34, 1:33, 1:33]
    return full.reshape(1, _D, _H, _W, _FILTERS)
